# baseline (speedup 1.0000x reference)
import functools

import jax
import jax.numpy as jnp
import numpy as np
from jax import lax
from jax.experimental import pallas as pl
from jax.experimental.pallas import tpu as pltpu
from jax.experimental.pallas import tpu_sc as plsc

N = 65536
NT = 16
CHUNK = N // NT
NV = CHUNK // 16
NBINS = 256
INT_MIN = np.int32(-2147483648)


def _sc_sort_unique(s):
    mesh = plsc.VectorSubcoreMesh(core_axis_name="c", subcore_axis_name="s")

    @functools.partial(
        pl.kernel,
        mesh=mesh,
        compiler_params=pltpu.CompilerParams(needs_layout_passes=False),
        out_type=[
            jax.ShapeDtypeStruct((N,), jnp.float32),
            jax.ShapeDtypeStruct((NT * NBINS,), jnp.int32),
            jax.ShapeDtypeStruct((NT, 16), jnp.int32),
        ],
        scratch_types=[
            pltpu.VMEM_SHARED((N,), jnp.float32),
            pltpu.VMEM_SHARED((N,), jnp.float32),
            pltpu.VMEM((CHUNK,), jnp.float32),
            pltpu.VMEM((CHUNK,), jnp.float32),
            pltpu.VMEM((32, 128), jnp.int32),
            pltpu.VMEM((CHUNK,), jnp.int32),
            pltpu.VMEM((NBINS,), jnp.int32),
            pltpu.VMEM((NBINS,), jnp.int32),
            pltpu.VMEM((NT * NBINS,), jnp.int32),
            pltpu.VMEM((16,), jnp.int32),
            pltpu.VMEM((NT, 16), jnp.int32),
            pltpu.VMEM((16,), jnp.float32),
            pltpu.SemaphoreType.DMA,
        ],
    )
    def sc_kernel(s_hbm, out_hbm, hsp, cnts_sp, spa, spb, inbuf, valb,
                  idx3d, encb, counters, histloc, scanbuf, cnt16, cntl,
                  pbuf, sem):
        cid = lax.axis_index("c")
        t = lax.axis_index("s")
        base = pl.multiple_of(t * CHUNK, CHUNK)
        iota = lax.iota(jnp.int32, 16)
        on0 = cid == 0

        def keybits(k):
            return plsc.bitcast(inbuf[pl.ds(k * 16, 16)], jnp.int32)

        def to_sortable(b):
            m = lax.shift_right_arithmetic(b, 31) | INT_MIN
            return b ^ m

        def from_sortable(u):
            return jnp.where(u < 0, u ^ INT_MIN, ~u)

        def digit_of(u, shift):
            uu = plsc.bitcast(u, jnp.uint32)
            return ((uu >> np.uint32(shift)) & np.uint32(0xFF)).astype(jnp.int32)

        def hist_phase(src, shift, first):
            @pl.when(on0)
            def _():
                with jax.named_scope("stage"):
                    pltpu.sync_copy(src.at[pl.ds(base, CHUNK)], inbuf)
                with jax.named_scope("hist"):
                    def zbody(k, c):
                        histloc[pl.ds(k * 16, 16)] = jnp.zeros((16,),
                                                               jnp.int32)
                        return c
                    lax.fori_loop(0, NBINS // 16, zbody, 0)

                    def hbody(k0, c):
                        for dk in range(4):
                            k = k0 * 4 + dk
                            b = keybits(k)
                            u = to_sortable(b) if first else b
                            d = digit_of(u, shift)
                            cnt, mlast = plsc.scan_count(d)
                            plsc.addupdate_scatter(histloc, [d],
                                                   cnt.astype(jnp.int32),
                                                   mask=mlast)
                        return c
                    lax.fori_loop(0, NV // 4, hbody, 0)
                    pltpu.sync_copy(
                        histloc,
                        hsp.at[pl.ds(pl.multiple_of(t * NBINS, NBINS),
                                     NBINS)])

        def perm_phase(dst, shift, first):
            @pl.when(on0)
            def _():
                with jax.named_scope("scan"):
                    pltpu.sync_copy(hsp, scanbuf)

                    def sbody(k, carry):
                        acc_all = jnp.zeros((16,), jnp.int32)
                        acc_lt = jnp.zeros((16,), jnp.int32)
                        for tp in range(NT):
                            h = scanbuf[pl.ds(tp * NBINS + k * 16, 16)]
                            acc_all = acc_all + h
                            acc_lt = acc_lt + jnp.where(t > tp, h, 0)
                        incl = plsc.cumsum(acc_all)
                        excl = incl - acc_all
                        counters[pl.ds(k * 16, 16)] = carry + excl + acc_lt
                        last = incl.at[jnp.full((16,), 15, jnp.int32)].get(
                            mode="promise_in_bounds")
                        return carry + last
                    lax.fori_loop(0, NBINS // 16, sbody,
                                  jnp.zeros((16,), jnp.int32))

                with jax.named_scope("perm"):
                    def pbody(k0, c):
                        for dk in range(4):
                            k = k0 * 4 + dk
                            b = keybits(k)
                            u = to_sortable(b) if first else b
                            d = digit_of(u, shift)
                            cnt, mlast = plsc.scan_count(d)
                            cnt = cnt.astype(jnp.int32)
                            basev = plsc.load_gather(counters, [d])
                            dest = basev + cnt - 1
                            plsc.addupdate_scatter(counters, [d], cnt,
                                                   mask=mlast)
                            valb[pl.ds(k * 16, 16)] = plsc.bitcast(
                                u, jnp.float32)
                            idx3d[k // 8, pl.ds((k % 8) * 16, 16)] = dest
                        return c
                    lax.fori_loop(0, NV // 4, pbody, 0)

                with jax.named_scope("scat"):
                    handles = [
                        pltpu.async_copy(valb.at[pl.ds(j * 128, 128)],
                                         dst.at[idx3d.at[j]], sem)
                        for j in range(32)
                    ]
                    for h in handles:
                        h.wait()

        def one_pass(src, dst, shift, first):
            with jax.named_scope("pass_s%d" % shift):
                hist_phase(src, shift, first)
                with jax.named_scope("bar1"):
                    plsc.subcore_barrier()
                perm_phase(dst, shift, first)
                with jax.named_scope("bar2"):
                    plsc.subcore_barrier()

        one_pass(s_hbm, spa, 0, True)
        one_pass(spa, spb, 8, False)
        one_pass(spb, spa, 16, False)
        one_pass(spa, spb, 24, False)

        @pl.when(on0)
        def _():
            with jax.named_scope("dedup"):
                poff = pl.multiple_of(jnp.maximum(base - 16, 0), 16)
                pltpu.sync_copy(spb.at[pl.ds(poff, 16)], pbuf)
                pltpu.sync_copy(spb.at[pl.ds(base, CHUNK)], inbuf)
                pb = plsc.bitcast(pbuf[...], jnp.int32)
                pb15 = pb.at[jnp.full((16,), 15, jnp.int32)].get(
                    mode="promise_in_bounds")
                v0 = keybits(0)
                v0f = v0.at[jnp.zeros((16,), jnp.int32)].get(
                    mode="promise_in_bounds")
                carry0 = jnp.where(t == 0, ~v0f, pb15)

                def fbody(k, carry):
                    prevlast, run = carry
                    b = keybits(k)
                    shifted = b.at[jnp.maximum(iota - 1, 0)].get(
                        mode="promise_in_bounds")
                    prev = jnp.where(iota == 0, prevlast, shifted)
                    mask = b != prev
                    c = plsc.cumsum(mask.astype(jnp.int32))
                    rank_incl = run + c
                    enc = jnp.where(mask, rank_incl - 1,
                                    -(k * 16 + iota - rank_incl) - 1)
                    encb[pl.ds(k * 16, 16)] = enc
                    valb[pl.ds(k * 16, 16)] = jnp.where(
                        mask, plsc.bitcast(from_sortable(b), jnp.float32),
                        np.float32(0.0))
                    newlast = b.at[jnp.full((16,), 15, jnp.int32)].get(
                        mode="promise_in_bounds")
                    newrun = rank_incl.at[jnp.full((16,), 15,
                                                   jnp.int32)].get(
                        mode="promise_in_bounds")
                    return (newlast, newrun)

                _, runf = lax.fori_loop(0, NV, fbody,
                                        (carry0, jnp.zeros((16,), jnp.int32)))
                cnt16[...] = runf
                pltpu.sync_copy(cnt16, cnts_sp.at[t])

        with jax.named_scope("bar3"):
            plsc.subcore_barrier()

        @pl.when(on0)
        def _():
            with jax.named_scope("outscat"):
                pltpu.sync_copy(cnts_sp, cntl)
                start = jnp.zeros((16,), jnp.int32)
                for tp in range(NT):
                    row = cntl[tp, pl.ds(0, 16)]
                    start = start + jnp.where(t > tp, row, 0)
                dupstart = t * CHUNK - start

                def gbody(k0, c):
                    for dk in range(4):
                        k = k0 * 4 + dk
                        e = encb[pl.ds(k * 16, 16)]
                        dest = jnp.where(e >= 0, start + e,
                                         (N - dupstart) + e)
                        idx3d[k // 8, pl.ds((k % 8) * 16, 16)] = dest
                    return c
                lax.fori_loop(0, NV // 4, gbody, 0)

                handles = [
                    pltpu.async_copy(valb.at[pl.ds(j * 128, 128)],
                                     out_hbm.at[idx3d.at[j]], sem)
                    for j in range(32)
                ]
                for h in handles:
                    h.wait()

    return sc_kernel(s)


def kernel(input_, location_):
    bits = location_.shape[2]
    exp = np.array([2.0 ** i for i in range(bits)], dtype=np.float32)
    s = jnp.sum(location_ * exp, axis=1).ravel()
    return _sc_sort_unique(s)[0]


# speedup vs baseline: 3.1568x; 3.1568x over previous
import functools

import jax
import jax.numpy as jnp
import numpy as np
from jax import lax
from jax.experimental import pallas as pl
from jax.experimental.pallas import tpu as pltpu
from jax.experimental.pallas import tpu_sc as plsc

N = 65536
NT = 16
CHUNK = N // NT
NV = CHUNK // 16
NBINS = 256
INT_MIN = np.int32(-2147483648)


def _sc_sort_unique(s):
    mesh = plsc.VectorSubcoreMesh(core_axis_name="c", subcore_axis_name="s")

    @functools.partial(
        pl.kernel,
        mesh=mesh,
        compiler_params=pltpu.CompilerParams(needs_layout_passes=False),
        out_type=[
            jax.ShapeDtypeStruct((N,), jnp.float32),
            jax.ShapeDtypeStruct((NT * NBINS,), jnp.int32),
            jax.ShapeDtypeStruct((NT, 16), jnp.int32),
        ],
        scratch_types=[
            pltpu.VMEM_SHARED((N,), jnp.float32),
            pltpu.VMEM_SHARED((N,), jnp.float32),
            pltpu.VMEM((CHUNK,), jnp.float32),
            pltpu.VMEM((CHUNK,), jnp.float32),
            pltpu.VMEM((32, 128), jnp.int32),
            pltpu.VMEM((CHUNK,), jnp.int32),
            pltpu.VMEM((NBINS,), jnp.int32),
            pltpu.VMEM((NBINS,), jnp.int32),
            pltpu.VMEM((NT * NBINS,), jnp.int32),
            pltpu.VMEM((16,), jnp.int32),
            pltpu.VMEM((NT, 16), jnp.int32),
            pltpu.VMEM((16,), jnp.float32),
            pltpu.SemaphoreType.DMA,
        ],
    )
    def sc_kernel(s_hbm, out_hbm, hsp, cnts_sp, spa, spb, inbuf, valb,
                  idx3d, encb, counters, histloc, scanbuf, cnt16, cntl,
                  pbuf, sem):
        cid = lax.axis_index("c")
        t = lax.axis_index("s")
        base = pl.multiple_of(t * CHUNK, CHUNK)
        iota = lax.iota(jnp.int32, 16)
        on0 = cid == 0

        def keybits(k):
            return plsc.bitcast(inbuf[pl.ds(k * 16, 16)], jnp.int32)

        def to_sortable(b):
            m = lax.shift_right_arithmetic(b, 31) | INT_MIN
            return b ^ m

        def from_sortable(u):
            return jnp.where(u < 0, u ^ INT_MIN, ~u)

        def digit_of(u, shift):
            uu = plsc.bitcast(u, jnp.uint32)
            return ((uu >> np.uint32(shift)) & np.uint32(0xFF)).astype(jnp.int32)

        def hist_phase(src, shift, first):
            @pl.when(on0)
            def _():
                with jax.named_scope("stage"):
                    pltpu.sync_copy(src.at[pl.ds(base, CHUNK)], inbuf)
                with jax.named_scope("hist"):
                    def zbody(k, c):
                        histloc[pl.ds(k * 16, 16)] = jnp.zeros((16,),
                                                               jnp.int32)
                        return c
                    lax.fori_loop(0, NBINS // 16, zbody, 0)

                    def hbody(k0, c):
                        for dk in range(4):
                            k = k0 * 4 + dk
                            b = keybits(k)
                            u = to_sortable(b) if first else b
                            d = digit_of(u, shift)
                            cnt, mlast = plsc.scan_count(d)
                            plsc.addupdate_scatter(histloc, [d],
                                                   cnt.astype(jnp.int32),
                                                   mask=mlast)
                        return c
                    lax.fori_loop(0, NV // 4, hbody, 0)
                    pltpu.sync_copy(
                        histloc,
                        hsp.at[pl.ds(pl.multiple_of(t * NBINS, NBINS),
                                     NBINS)])

        def perm_phase(dst, shift, first):
            @pl.when(on0)
            def _():
                with jax.named_scope("scan"):
                    pltpu.sync_copy(hsp, scanbuf)

                    def sbody(k, carry):
                        acc_all = jnp.zeros((16,), jnp.int32)
                        acc_lt = jnp.zeros((16,), jnp.int32)
                        for tp in range(NT):
                            h = scanbuf[pl.ds(tp * NBINS + k * 16, 16)]
                            acc_all = acc_all + h
                            acc_lt = acc_lt + jnp.where(t > tp, h, 0)
                        incl = plsc.cumsum(acc_all)
                        excl = incl - acc_all
                        counters[pl.ds(k * 16, 16)] = carry + excl + acc_lt
                        last = incl.at[jnp.full((16,), 15, jnp.int32)].get(
                            mode="promise_in_bounds")
                        return carry + last
                    lax.fori_loop(0, NBINS // 16, sbody,
                                  jnp.zeros((16,), jnp.int32))

                with jax.named_scope("perm"):
                    def pbody(k0, c):
                        for dk in range(4):
                            k = k0 * 4 + dk
                            b = keybits(k)
                            u = to_sortable(b) if first else b
                            d = digit_of(u, shift)
                            cnt, mlast = plsc.scan_count(d)
                            cnt = cnt.astype(jnp.int32)
                            basev = plsc.load_gather(counters, [d])
                            dest = basev + cnt - 1
                            plsc.addupdate_scatter(counters, [d], cnt,
                                                   mask=mlast)
                            valb[pl.ds(k * 16, 16)] = plsc.bitcast(
                                u, jnp.float32)
                            idx3d[k // 8, pl.ds((k % 8) * 16, 16)] = dest
                        return c
                    lax.fori_loop(0, NV // 4, pbody, 0)

                with jax.named_scope("scat"):
                    handles = [
                        pltpu.async_copy(valb.at[pl.ds(j * 128, 128)],
                                         dst.at[idx3d.at[j]], sem)
                        for j in range(32)
                    ]
                    for h in handles:
                        h.wait()

        def one_pass(src, dst, shift, first):
            with jax.named_scope("pass_s%d" % shift):
                hist_phase(src, shift, first)
                with jax.named_scope("bar1"):
                    plsc.subcore_barrier()
                perm_phase(dst, shift, first)
                with jax.named_scope("bar2"):
                    plsc.subcore_barrier()

        one_pass(s_hbm, spa, 0, True)
        one_pass(spa, spb, 8, False)
        one_pass(spb, spa, 16, False)
        one_pass(spa, spb, 24, False)

        @pl.when(on0)
        def _():
            with jax.named_scope("dedup"):
                poff = pl.multiple_of(jnp.maximum(base - 16, 0), 16)
                pltpu.sync_copy(spb.at[pl.ds(poff, 16)], pbuf)
                pltpu.sync_copy(spb.at[pl.ds(base, CHUNK)], inbuf)
                pb = plsc.bitcast(pbuf[...], jnp.int32)
                pb15 = pb.at[jnp.full((16,), 15, jnp.int32)].get(
                    mode="promise_in_bounds")
                v0 = keybits(0)
                v0f = v0.at[jnp.zeros((16,), jnp.int32)].get(
                    mode="promise_in_bounds")
                carry0 = jnp.where(t == 0, ~v0f, pb15)

                def fbody(k, carry):
                    prevlast, run = carry
                    b = keybits(k)
                    shifted = b.at[jnp.maximum(iota - 1, 0)].get(
                        mode="promise_in_bounds")
                    prev = jnp.where(iota == 0, prevlast, shifted)
                    mask = b != prev
                    c = plsc.cumsum(mask.astype(jnp.int32))
                    rank_incl = run + c
                    enc = jnp.where(mask, rank_incl - 1,
                                    -(k * 16 + iota - rank_incl) - 1)
                    encb[pl.ds(k * 16, 16)] = enc
                    valb[pl.ds(k * 16, 16)] = jnp.where(
                        mask, plsc.bitcast(from_sortable(b), jnp.float32),
                        np.float32(0.0))
                    newlast = b.at[jnp.full((16,), 15, jnp.int32)].get(
                        mode="promise_in_bounds")
                    newrun = rank_incl.at[jnp.full((16,), 15,
                                                   jnp.int32)].get(
                        mode="promise_in_bounds")
                    return (newlast, newrun)

                _, runf = lax.fori_loop(0, NV, fbody,
                                        (carry0, jnp.zeros((16,), jnp.int32)))
                cnt16[...] = runf
                pltpu.sync_copy(cnt16, cnts_sp.at[t])

        plsc.subcore_barrier()

        @pl.when(on0)
        def _():
            with jax.named_scope("outscat"):
                pltpu.sync_copy(cnts_sp, cntl)
                start_v = jnp.zeros((16,), jnp.int32)
                for tp in range(NT):
                    row = cntl[tp, pl.ds(0, 16)]
                    start_v = start_v + jnp.where(t > tp, row, 0)
                dupstart = t * CHUNK - start_v

                def gbody(k0, c):
                    for dk in range(4):
                        k = k0 * 4 + dk
                        e = encb[pl.ds(k * 16, 16)]
                        dest = jnp.where(e >= 0, start_v + e,
                                         (N - dupstart) + e)
                        idx3d[k // 8, pl.ds((k % 8) * 16, 16)] = dest
                    return c
                lax.fori_loop(0, NV // 4, gbody, 0)

                handles = [
                    pltpu.async_copy(valb.at[pl.ds(j * 128, 128)],
                                     spa.at[idx3d.at[j]], sem)
                    for j in range(32)
                ]
                for h in handles:
                    h.wait()

        plsc.subcore_barrier()

        @pl.when(on0)
        def _():
            with jax.named_scope("outcopy"):
                pltpu.sync_copy(spa.at[pl.ds(base, CHUNK)], inbuf)
                pltpu.sync_copy(inbuf, out_hbm.at[pl.ds(base, CHUNK)])

    return sc_kernel(s)


def kernel(input_, location_):
    bits = location_.shape[2]
    exp = np.array([2.0 ** i for i in range(bits)], dtype=np.float32)
    s = jnp.sum(location_ * exp, axis=1).ravel()
    return _sc_sort_unique(s)[0]
